# Initial kernel scaffold; baseline (speedup 1.0000x reference)
import jax
import jax.numpy as jnp
from jax.experimental import pallas as pl


def kernel(x, edge_index, edge_types, Wn, bn, W1, b1, W2, b2, W3, b3, W4, b4, W2b, b2b, W3b, b3b, W4b, b4b, Wo1, bo1, Wo2, bo2):
    raise NotImplementedError("write your pallas kernel here")


# speedup vs baseline: 11.6638x; 11.6638x over previous
import functools

import jax
import jax.numpy as jnp
from jax import lax
from jax.experimental import pallas as pl
from jax.experimental.pallas import tpu as pltpu
from jax.experimental.pallas import tpu_sc as plsc

N = 50000
E = 800000
D = 64
H = 32
NCONV = 7
DEPTH = 4
GDEG = 50

NC = 2
NS = 16
LANES = 16

EB = 64
KB = 128
EP = 800768
NPAD = 50048
ROWS_PER_TILE = NPAD // NS
ZROWS = 184

COLMAP = (0, 1, 2, 3, 5, 6, 7)

_mesh = plsc.VectorSubcoreMesh(core_axis_name="c", subcore_axis_name="s")
_sc_params = pltpu.CompilerParams(needs_layout_passes=False)
_sc_params_nt = pltpu.CompilerParams(needs_layout_passes=False,
                                     use_tc_tiling_on_sc=False)


@functools.partial(
    pl.kernel,
    out_type=jax.ShapeDtypeStruct((NC, NPAD, 16), jnp.float32),
    mesh=_mesh,
    compiler_params=_sc_params_nt,
    scratch_types=[
        pltpu.VMEM((EB,), jnp.int32),
        pltpu.VMEM((EB,), jnp.int32),
        pltpu.VMEM((EB,), jnp.int32),
        pltpu.VMEM((2 * EB, 16), jnp.float32),
        pltpu.VMEM((2 * EB,), jnp.int32),
        pltpu.VMEM((ZROWS, 16), jnp.float32),
        pltpu.VMEM_SHARED((NPAD, 16), jnp.float32),
    ],
)
def _sc_degrees(srcp_h, dstp_h, etyp_h, degp_h, ebS, ebD, ebT, oh, oidx, zb, dacc):
    cc = lax.axis_index("c")
    ss = lax.axis_index("s")
    w = cc * NS + ss

    @pl.loop(0, ZROWS)
    def _(i):
        zb[i, :] = jnp.zeros((LANES,), jnp.float32)

    @pl.loop(0, ROWS_PER_TILE // ZROWS)
    def _(i):
        pltpu.sync_copy(zb, dacc.at[pl.ds(ss * ROWS_PER_TILE + i * ZROWS, ZROWS)])

    @pl.loop(0, 2 * EB)
    def _(i):
        oh[i, :] = jnp.zeros((LANES,), jnp.float32)

    plsc.subcore_barrier()

    base0 = w * (EP // 32)

    @pl.loop(0, EP // 32 // EB)
    def _(bi):
        zero16 = jnp.zeros((LANES,), jnp.float32)
        iota = lax.iota(jnp.int32, LANES)
        off = base0 + bi * EB
        pltpu.sync_copy(srcp_h.at[pl.ds(off, EB)], ebS)
        pltpu.sync_copy(dstp_h.at[pl.ds(off, EB)], ebD)
        pltpu.sync_copy(etyp_h.at[pl.ds(off, EB)], ebT)
        for q in range(EB // LANES):
            j = q * LANES + iota
            t = ebT[pl.ds(q * LANES, LANES)]
            d = ebD[pl.ds(q * LANES, LANES)]
            s_ = ebS[pl.ds(q * LANES, LANES)]
            val = ((off + j) < E).astype(jnp.float32)
            plsc.store_scatter(oh, [2 * j, t - 1], val)
            plsc.store_scatter(oh, [2 * j + 1, t + 3], val)
            plsc.store_scatter(oidx, [2 * j], d)
            plsc.store_scatter(oidx, [2 * j + 1], s_)
        pltpu.sync_copy(oh, dacc.at[oidx], add=True)
        for q in range(EB // LANES):
            j = q * LANES + iota
            t = ebT[pl.ds(q * LANES, LANES)]
            plsc.store_scatter(oh, [2 * j, t - 1], zero16)
            plsc.store_scatter(oh, [2 * j + 1, t + 3], zero16)

    plsc.subcore_barrier()
    pltpu.sync_copy(
        dacc.at[pl.ds(ss * ROWS_PER_TILE, ROWS_PER_TILE)],
        degp_h.at[cc, pl.ds(ss * ROWS_PER_TILE, ROWS_PER_TILE)],
    )


@functools.partial(
    pl.kernel,
    out_type=(
        jax.ShapeDtypeStruct((2 * EP,), jnp.int32),
        jax.ShapeDtypeStruct((2 * EP,), jnp.float32),
        jax.ShapeDtypeStruct((2 * EP,), jnp.int32),
    ),
    mesh=_mesh,
    compiler_params=_sc_params_nt,
    scratch_types=[
        pltpu.VMEM((EB,), jnp.int32),
        pltpu.VMEM((EB,), jnp.int32),
        pltpu.VMEM((EB,), jnp.int32),
        pltpu.VMEM((EB, 16), jnp.float32),
        pltpu.VMEM((EB, 16), jnp.float32),
        pltpu.VMEM((EB,), jnp.int32),
        pltpu.VMEM((EB,), jnp.int32),
        pltpu.VMEM((EB,), jnp.float32),
        pltpu.VMEM((EB,), jnp.float32),
        pltpu.SemaphoreType.DMA,
        pltpu.SemaphoreType.DMA,
    ],
)
def _sc_norms(srcp_h, dstp_h, etyp_h, dis_h, gix_h, scl_h, six_h,
              ebS, ebD, ebT, rs, rd, gof, gob, sof, sob, sem1, sem2):
    cc = lax.axis_index("c")
    ss = lax.axis_index("s")
    w = cc * NS + ss
    base0 = w * (EP // 32)

    @pl.loop(0, EP // 32 // EB)
    def _(bi):
        iota = lax.iota(jnp.int32, LANES)
        off = base0 + bi * EB
        pltpu.sync_copy(srcp_h.at[pl.ds(off, EB)], ebS)
        pltpu.sync_copy(dstp_h.at[pl.ds(off, EB)], ebD)
        pltpu.sync_copy(etyp_h.at[pl.ds(off, EB)], ebT)
        cp1 = pltpu.async_copy(dis_h.at[ebS], rs, sem1)
        cp2 = pltpu.async_copy(dis_h.at[ebD], rd, sem2)
        cp1.wait()
        cp2.wait()
        for q in range(EB // LANES):
            j = q * LANES + iota
            t = ebT[pl.ds(q * LANES, LANES)]
            s_ = ebS[pl.ds(q * LANES, LANES)]
            d = ebD[pl.ds(q * LANES, LANES)]
            live_f = (off + j) < E
            a = plsc.load_gather(rs, [j, t - 1])
            b = plsc.load_gather(rd, [j, t - 1])
            sf = jnp.where(live_f, a * b, jnp.zeros_like(a))
            c3 = plsc.load_gather(rs, [j, t + 3])
            d3 = plsc.load_gather(rd, [j, t + 3])
            live_b = live_f & (t != 1)
            sb = jnp.where(live_b, c3 * d3, jnp.zeros_like(a))
            gf = (t - 1) * N + s_
            gb = jnp.where(t != 1, (t + 2) * N + d, jnp.zeros_like(d))
            gof[pl.ds(q * LANES, LANES)] = gf
            gob[pl.ds(q * LANES, LANES)] = gb
            sof[pl.ds(q * LANES, LANES)] = sf
            sob[pl.ds(q * LANES, LANES)] = sb
        pltpu.sync_copy(gof, gix_h.at[pl.ds(off, EB)])
        pltpu.sync_copy(gob, gix_h.at[pl.ds(EP + off, EB)])
        pltpu.sync_copy(sof, scl_h.at[pl.ds(off, EB)])
        pltpu.sync_copy(sob, scl_h.at[pl.ds(EP + off, EB)])
        pltpu.sync_copy(ebD, six_h.at[pl.ds(off, EB)])
        pltpu.sync_copy(ebS, six_h.at[pl.ds(EP + off, EB)])


@functools.partial(
    pl.kernel,
    out_type=(
        jax.ShapeDtypeStruct((NPAD, H), jnp.float32),
        jax.ShapeDtypeStruct((NPAD, H), jnp.float32),
    ),
    mesh=_mesh,
    compiler_params=_sc_params_nt,
    scratch_types=[
        pltpu.VMEM((KB,), jnp.int32),
        pltpu.VMEM((KB,), jnp.float32),
        pltpu.VMEM((KB,), jnp.int32),
        pltpu.VMEM((KB, H), jnp.float32),
        pltpu.VMEM((ZROWS, H), jnp.float32),
        pltpu.VMEM_SHARED((NPAD, H), jnp.float32),
        pltpu.SemaphoreType.DMA,
    ],
)
def _sc_agg(hla_h, hlb_h, gix_h, scl_h, six_h, agg_a_h, agg_b_h,
            gbuf, sbuf, ibuf, rows, zb, acc, sem):
    cc = lax.axis_index("c")
    ss = lax.axis_index("s")

    @pl.loop(0, ZROWS)
    def _(i):
        zero16 = jnp.zeros((LANES,), jnp.float32)
        zb[i, pl.ds(0, LANES)] = zero16
        zb[i, pl.ds(LANES, LANES)] = zero16

    @pl.loop(0, ROWS_PER_TILE // ZROWS)
    def _(i):
        pltpu.sync_copy(zb, acc.at[pl.ds(ss * ROWS_PER_TILE + i * ZROWS, ZROWS)])

    plsc.subcore_barrier()

    ebase = ss * (2 * EP // NS)

    @pl.loop(0, 2 * EP // NS // KB)
    def _(bi):
        off = ebase + bi * KB
        pltpu.sync_copy(gix_h.at[pl.ds(off, KB)], gbuf)
        pltpu.sync_copy(scl_h.at[pl.ds(off, KB)], sbuf)
        pltpu.sync_copy(six_h.at[pl.ds(off, KB)], ibuf)

        @pl.when(cc == 0)
        def _():
            pltpu.async_copy(hla_h.at[gbuf], rows, sem).wait()

        @pl.when(cc == 1)
        def _():
            pltpu.async_copy(hlb_h.at[gbuf], rows, sem).wait()

        @pl.loop(0, KB // LANES)
        def _(b):
            s16 = sbuf[pl.ds(b * LANES, LANES)]
            for r in range(LANES):
                j = b * LANES + r
                s = s16[r]
                rows[j, pl.ds(0, LANES)] = rows[j, pl.ds(0, LANES)] * s
                rows[j, pl.ds(LANES, LANES)] = rows[j, pl.ds(LANES, LANES)] * s

        pltpu.sync_copy(rows, acc.at[ibuf], add=True)

    plsc.subcore_barrier()

    @pl.when(cc == 0)
    def _():
        pltpu.sync_copy(acc.at[pl.ds(ss * ROWS_PER_TILE, ROWS_PER_TILE)],
                        agg_a_h.at[pl.ds(ss * ROWS_PER_TILE, ROWS_PER_TILE)])

    @pl.when(cc == 1)
    def _():
        pltpu.sync_copy(acc.at[pl.ds(ss * ROWS_PER_TILE, ROWS_PER_TILE)],
                        agg_b_h.at[pl.ds(ss * ROWS_PER_TILE, ROWS_PER_TILE)])


RB = 1000


def _tc_deg_finish(degp):
    def body(degp_ref, dis_ref, inv_ref):
        dsum = degp_ref[0] + degp_ref[1] + 1.0
        dis_ref[...] = lax.rsqrt(dsum)
        inv_ref[...] = 1.0 / dsum

    return pl.pallas_call(
        body,
        grid=(N // RB,),
        in_specs=[pl.BlockSpec((NC, RB, 16), lambda i: (0, i, 0))],
        out_specs=(pl.BlockSpec((RB, 16), lambda i: (i, 0)),
                   pl.BlockSpec((RB, 16), lambda i: (i, 0))),
        out_shape=(jax.ShapeDtypeStruct((N, 16), jnp.float32),
                   jax.ShapeDtypeStruct((N, 16), jnp.float32)),
    )(degp)


def _tc_first_matmul(x, wnT, bn2, wa, wb):
    def body(x_ref, wn_ref, bn_ref, wa_ref, wb_ref, hla_ref, hlb_ref):
        hh = x_ref[...] * wn_ref[...] + bn_ref[...]
        for c in range(NCONV):
            hla_ref[c] = jnp.dot(hh, wa_ref[c], preferred_element_type=jnp.float32)
            hlb_ref[c] = jnp.dot(hh, wb_ref[c], preferred_element_type=jnp.float32)

    return pl.pallas_call(
        body,
        grid=(N // RB,),
        in_specs=[
            pl.BlockSpec((RB, 1), lambda i: (i, 0)),
            pl.BlockSpec((1, D), lambda i: (0, 0)),
            pl.BlockSpec((1, D), lambda i: (0, 0)),
            pl.BlockSpec((NCONV, D, H), lambda i: (0, 0, 0)),
            pl.BlockSpec((NCONV, D, H), lambda i: (0, 0, 0)),
        ],
        out_specs=(pl.BlockSpec((NCONV, RB, H), lambda i: (0, i, 0)),
                   pl.BlockSpec((NCONV, RB, H), lambda i: (0, i, 0))),
        out_shape=(jax.ShapeDtypeStruct((NCONV, N, H), jnp.float32),
                   jax.ShapeDtypeStruct((NCONV, N, H), jnp.float32)),
    )(x, wnT, bn2, wa, wb)


def _combine_block(agg_a_ref, agg_b_ref, hla_ref, hlb_ref, inv_ref, bsum_ref):
    sa = agg_a_ref[...]
    sb = agg_b_ref[...]
    invv = inv_ref[...]
    for c in range(NCONV):
        iv = invv[:, COLMAP[c]:COLMAP[c] + 1]
        sa = sa + iv * hla_ref[c]
        sb = sb + iv * hlb_ref[c]
    hh = jnp.concatenate([sa, sb], axis=1) + bsum_ref[...]
    return jnp.maximum(hh, 0.0)


def _tc_fused_layer(agg_a, agg_b, hla, hlb, inv, bsum2, wa, wb):
    def body(agg_a_ref, agg_b_ref, hla_ref, hlb_ref, inv_ref, bsum_ref,
             wa_ref, wb_ref, hla_n, hlb_n):
        hh = _combine_block(agg_a_ref, agg_b_ref, hla_ref, hlb_ref, inv_ref, bsum_ref)
        for c in range(NCONV):
            hla_n[c] = jnp.dot(hh, wa_ref[c], preferred_element_type=jnp.float32)
            hlb_n[c] = jnp.dot(hh, wb_ref[c], preferred_element_type=jnp.float32)

    return pl.pallas_call(
        body,
        grid=(N // RB,),
        in_specs=[
            pl.BlockSpec((RB, H), lambda i: (i, 0)),
            pl.BlockSpec((RB, H), lambda i: (i, 0)),
            pl.BlockSpec((NCONV, RB, H), lambda i: (0, i, 0)),
            pl.BlockSpec((NCONV, RB, H), lambda i: (0, i, 0)),
            pl.BlockSpec((RB, 16), lambda i: (i, 0)),
            pl.BlockSpec((1, D), lambda i: (0, 0)),
            pl.BlockSpec((NCONV, D, H), lambda i: (0, 0, 0)),
            pl.BlockSpec((NCONV, D, H), lambda i: (0, 0, 0)),
        ],
        out_specs=(pl.BlockSpec((NCONV, RB, H), lambda i: (0, i, 0)),
                   pl.BlockSpec((NCONV, RB, H), lambda i: (0, i, 0))),
        out_shape=(jax.ShapeDtypeStruct((NCONV, N, H), jnp.float32),
                   jax.ShapeDtypeStruct((NCONV, N, H), jnp.float32)),
    )(agg_a, agg_b, hla, hlb, inv, bsum2, wa, wb)


def _tc_final_combine(agg_a, agg_b, hla, hlb, inv, bsum2):
    def body(agg_a_ref, agg_b_ref, hla_ref, hlb_ref, inv_ref, bsum_ref, h_ref):
        h_ref[...] = _combine_block(agg_a_ref, agg_b_ref, hla_ref, hlb_ref,
                                    inv_ref, bsum_ref)

    return pl.pallas_call(
        body,
        grid=(N // RB,),
        in_specs=[
            pl.BlockSpec((RB, H), lambda i: (i, 0)),
            pl.BlockSpec((RB, H), lambda i: (i, 0)),
            pl.BlockSpec((NCONV, RB, H), lambda i: (0, i, 0)),
            pl.BlockSpec((NCONV, RB, H), lambda i: (0, i, 0)),
            pl.BlockSpec((RB, 16), lambda i: (i, 0)),
            pl.BlockSpec((1, D), lambda i: (0, 0)),
        ],
        out_specs=pl.BlockSpec((RB, D), lambda i: (i, 0)),
        out_shape=jax.ShapeDtypeStruct((N, D), jnp.float32),
    )(agg_a, agg_b, hla, hlb, inv, bsum2)


GB = 8


def _tc_readout(h, wo1T, bo1_2, wo2T, bo2_2):
    def body(h_ref, wo1_ref, bo1_ref, wo2_ref, bo2_ref, out_ref):
        m = jnp.max(h_ref[...].reshape(GB, GDEG, D), axis=1)
        y = jnp.dot(m, wo1_ref[...], preferred_element_type=jnp.float32) + bo1_ref[...]
        y = jnp.maximum(y, 0.0)
        out_ref[...] = jnp.dot(y, wo2_ref[...], preferred_element_type=jnp.float32) + bo2_ref[...]

    ngraph = N // GDEG
    return pl.pallas_call(
        body,
        grid=(ngraph // GB,),
        in_specs=[
            pl.BlockSpec((GB * GDEG, D), lambda i: (i, 0)),
            pl.BlockSpec((D, D), lambda i: (0, 0)),
            pl.BlockSpec((1, D), lambda i: (0, 0)),
            pl.BlockSpec((D, 1), lambda i: (0, 0)),
            pl.BlockSpec((1, 1), lambda i: (0, 0)),
        ],
        out_specs=pl.BlockSpec((GB, 1), lambda i: (i, 0)),
        out_shape=jax.ShapeDtypeStruct((ngraph, 1), jnp.float32),
    )(h, wo1T, bo1_2, wo2T, bo2_2)


def kernel(x, edge_index, edge_types, Wn, bn, W1, b1, W2, b2, W3, b3, W4, b4,
           W2b, b2b, W3b, b3b, W4b, b4b, Wo1, bo1, Wo2, bo2):
    src = edge_index[0]
    dst = edge_index[1]
    pad = EP - E
    srcp = jnp.concatenate([src, jnp.zeros((pad,), jnp.int32)])
    dstp = jnp.concatenate([dst, jnp.zeros((pad,), jnp.int32)])
    etyp = jnp.concatenate([edge_types, jnp.ones((pad,), jnp.int32)])

    degp = _sc_degrees(srcp, dstp, etyp)
    dis, inv = _tc_deg_finish(degp)
    gix, scl, six = _sc_norms(srcp, dstp, etyp, dis)

    Ws = [W1, W2, W3, W4, W2b, W3b, W4b]
    wa = jnp.stack([w[:H, :].T for w in Ws])
    wb = jnp.stack([w[H:, :].T for w in Ws])
    bsum2 = (b1 + b2 + b3 + b4 + b2b + b3b + b4b).reshape(1, D)

    hla, hlb = _tc_first_matmul(x, Wn.T, bn.reshape(1, D), wa, wb)
    for l in range(DEPTH):
        agg_a, agg_b = _sc_agg(hla.reshape(NCONV * N, H), hlb.reshape(NCONV * N, H),
                               gix, scl, six)
        if l < DEPTH - 1:
            hla, hlb = _tc_fused_layer(agg_a, agg_b, hla, hlb, inv, bsum2, wa, wb)
        else:
            h = _tc_final_combine(agg_a, agg_b, hla, hlb, inv, bsum2)

    return _tc_readout(h, Wo1.T, bo1.reshape(1, D), Wo2.T, bo2.reshape(1, 1))
